# baseline (speedup 1.0000x reference)
import functools

import jax
import jax.numpy as jnp
from jax import lax
from jax.experimental import pallas as pl
from jax.experimental.pallas import tpu as pltpu
from jax.experimental.pallas import tpu_sc as plsc

_NC, _NS, _L = 2, 16, 16
_NW = _NC * _NS
_CH = 16
_RIN = 1024
_ROUT = 256


def _sc_pool_body(x_hbm, o_hbm, inb, outb):
    c = lax.axis_index("c")
    s = lax.axis_index("s")
    wid = s * _NC + c
    n2 = o_hbm.shape[0] // _ROUT
    rows_per_w = n2 // _NW
    base = wid * rows_per_w
    nch = rows_per_w // _CH
    idx0 = lax.iota(jnp.int32, 16) * 2

    def chunk_body(g, carry):
        rbase = base + g * _CH
        pltpu.sync_copy(x_hbm.at[pl.ds(rbase * _RIN, _CH * _RIN)], inb)

        def row_body(r, carry2):
            rb = r * _RIN
            ob = r * _ROUT

            def vec_body(k, carry3):
                i_e = rb + 32 * k + idx0
                e1 = plsc.load_gather(inb, [i_e])
                o1 = plsc.load_gather(inb, [i_e + 1])
                e2 = plsc.load_gather(inb, [i_e + 512])
                o2 = plsc.load_gather(inb, [i_e + 513])
                e1 = jnp.maximum(e1, 0.0)
                o1 = jnp.maximum(o1, 0.0)
                e2 = jnp.maximum(e2, 0.0)
                o2 = jnp.maximum(o2, 0.0)
                den = (e1 + o1) + (e2 + o2)
                num = (e1 * e1 + o1 * o1) + (e2 * e2 + o2 * o2)
                res = num / jnp.where(den == 0.0, 1.0, den)
                outb[pl.ds(ob + 16 * k, 16)] = res
                return carry3

            return lax.fori_loop(0, 16, vec_body, carry2)

        lax.fori_loop(0, _CH, row_body, 0)
        pltpu.sync_copy(outb, o_hbm.at[pl.ds(rbase * _ROUT, _CH * _ROUT)])
        return carry

    lax.fori_loop(0, nch, chunk_body, 0)


def kernel(tensor):
    B, C, H, W = tensor.shape
    n2 = B * C * (H // 2)
    x = tensor.reshape(n2 * 2 * W)
    sc_pool = functools.partial(
        pl.kernel,
        out_type=jax.ShapeDtypeStruct((n2 * (W // 2),), jnp.float32),
        mesh=plsc.VectorSubcoreMesh(core_axis_name="c", subcore_axis_name="s"),
        compiler_params=pltpu.CompilerParams(needs_layout_passes=False),
        scratch_types=[
            pltpu.VMEM((_CH * _RIN,), jnp.float32),
            pltpu.VMEM((_CH * _ROUT,), jnp.float32),
        ],
    )(_sc_pool_body)
    out = sc_pool(x)
    return out.reshape(B, C, H // 2, W // 2)


# speedup vs baseline: 1.2677x; 1.2677x over previous
import functools

import jax
import jax.numpy as jnp
from jax import lax
from jax.experimental import pallas as pl
from jax.experimental.pallas import tpu as pltpu
from jax.experimental.pallas import tpu_sc as plsc

_NC, _NS, _L = 2, 16, 16
_NW = _NC * _NS
_CH = 32
_RIN = 1024
_ROUT = 256


def _compute_chunk(inb, outb, idx0):
    def row_body(r, carry):
        rb = r * _RIN
        ob = r * _ROUT
        for k in range(16):
            i_e = rb + 32 * k + idx0
            e1 = plsc.load_gather(inb, [i_e])
            o1 = plsc.load_gather(inb, [i_e + 1])
            e2 = plsc.load_gather(inb, [i_e + 512])
            o2 = plsc.load_gather(inb, [i_e + 513])
            e1 = jnp.maximum(e1, 0.0)
            o1 = jnp.maximum(o1, 0.0)
            e2 = jnp.maximum(e2, 0.0)
            o2 = jnp.maximum(o2, 0.0)
            den = (e1 + o1) + (e2 + o2)
            num = (e1 * e1 + o1 * o1) + (e2 * e2 + o2 * o2)
            res = num / jnp.where(den == 0.0, 1.0, den)
            outb[pl.ds(ob + 16 * k, 16)] = res
        return carry

    lax.fori_loop(0, _CH, row_body, 0)


def _sc_pool_body(x_hbm, o_hbm, ib0, ib1, ob0, ob1,
                  isem0, isem1, osem0, osem1):
    c = lax.axis_index("c")
    s = lax.axis_index("s")
    wid = s * _NC + c
    n2 = o_hbm.shape[0] // _ROUT
    rows_per_w = n2 // _NW
    base = wid * rows_per_w
    nch = rows_per_w // _CH
    last = nch - 1
    idx0 = lax.iota(jnp.int32, 16) * 2
    inbufs = (ib0, ib1)
    outbufs = (ob0, ob1)
    isems = (isem0, isem1)
    osems = (osem0, osem1)

    def in_src(g):
        return x_hbm.at[pl.ds((base + g * _CH) * _RIN, _CH * _RIN)]

    def out_dst(g):
        return o_hbm.at[pl.ds((base + g * _CH) * _ROUT, _CH * _ROUT)]

    for b in range(2):
        pltpu.make_async_copy(in_src(jnp.int32(b)), inbufs[b], isems[b]).start()

    def pair_body(g0, carry):
        for b in range(2):
            g = 2 * g0 + b
            pltpu.make_async_copy(in_src(g), inbufs[b], isems[b]).wait()

            @pl.when(g >= 2)
            def _():
                pltpu.make_async_copy(outbufs[b], out_dst(g), osems[b]).wait()

            _compute_chunk(inbufs[b], outbufs[b], idx0)
            pltpu.make_async_copy(outbufs[b], out_dst(g), osems[b]).start()
            gn = jnp.minimum(g + 2, last)
            pltpu.make_async_copy(in_src(gn), inbufs[b], isems[b]).start()
        return carry

    lax.fori_loop(0, nch // 2 - 1, pair_body, 0)

    for b in range(2):
        g = nch - 2 + b
        pltpu.make_async_copy(in_src(g), inbufs[b], isems[b]).wait()
        pltpu.make_async_copy(outbufs[b], out_dst(g), osems[b]).wait()
        _compute_chunk(inbufs[b], outbufs[b], idx0)
        pltpu.make_async_copy(outbufs[b], out_dst(g), osems[b]).start()
    for b in range(2):
        g = nch - 2 + b
        pltpu.make_async_copy(outbufs[b], out_dst(g), osems[b]).wait()


def kernel(tensor):
    B, C, H, W = tensor.shape
    n2 = B * C * (H // 2)
    x = tensor.reshape(n2 * 2 * W)
    sc_pool = functools.partial(
        pl.kernel,
        out_type=jax.ShapeDtypeStruct((n2 * (W // 2),), jnp.float32),
        mesh=plsc.VectorSubcoreMesh(core_axis_name="c", subcore_axis_name="s"),
        compiler_params=pltpu.CompilerParams(needs_layout_passes=False),
        scratch_types=[
            pltpu.VMEM((_CH * _RIN,), jnp.float32),
            pltpu.VMEM((_CH * _RIN,), jnp.float32),
            pltpu.VMEM((_CH * _ROUT,), jnp.float32),
            pltpu.VMEM((_CH * _ROUT,), jnp.float32),
            pltpu.SemaphoreType.DMA,
            pltpu.SemaphoreType.DMA,
            pltpu.SemaphoreType.DMA,
            pltpu.SemaphoreType.DMA,
        ],
    )(_sc_pool_body)
    out = sc_pool(x)
    return out.reshape(B, C, H // 2, W // 2)


# speedup vs baseline: 2.1572x; 1.7016x over previous
import functools

import jax
import jax.numpy as jnp
from jax import lax
from jax.experimental import pallas as pl
from jax.experimental.pallas import tpu as pltpu
from jax.experimental.pallas import tpu_sc as plsc

_NC, _NS, _L = 2, 16, 16
_NW = _NC * _NS
_CH = 32
_RIN = 1024
_ROUT = 256


def _compute_chunk(inb, outb, idx0):
    i_e1 = idx0
    i_o1 = idx0 + 1
    i_e2 = idx0 + 512
    i_o2 = idx0 + 513
    def row_body(r, carry):
        rb = r * _RIN
        ob = r * _ROUT
        for kk in range(0, 16, 4):
            ks = range(kk, kk + 4)
            wins = [inb.at[pl.ds(rb + 32 * k, 544)] for k in ks]
            g = [[jnp.maximum(plsc.load_gather(w, [i]), 0.0)
                  for i in (i_e1, i_o1, i_e2, i_o2)] for w in wins]
            dens = [(e1 + o1) + (e2 + o2) for e1, o1, e2, o2 in g]
            dens = [jnp.where(d == 0.0, 1.0, d) for d in dens]
            nums = [(e1 * e1 + o1 * o1) + (e2 * e2 + o2 * o2)
                    for e1, o1, e2, o2 in g]
            for k, num, den in zip(ks, nums, dens):
                outb[pl.ds(ob + 16 * k, 16)] = num / den
        return carry

    lax.fori_loop(0, _CH, row_body, 0)


def _sc_pool_body(x_hbm, o_hbm, ib0, ib1, ob0, ob1,
                  isem0, isem1, osem0, osem1):
    c = lax.axis_index("c")
    s = lax.axis_index("s")
    wid = s * _NC + c
    n2 = o_hbm.shape[0] // _ROUT
    rows_per_w = n2 // _NW
    base = wid * rows_per_w
    nch = rows_per_w // _CH
    last = nch - 1
    idx0 = lax.iota(jnp.int32, 16) * 2
    inbufs = (ib0, ib1)
    outbufs = (ob0, ob1)
    isems = (isem0, isem1)
    osems = (osem0, osem1)

    def in_src(g):
        return x_hbm.at[pl.ds((base + g * _CH) * _RIN, _CH * _RIN)]

    def out_dst(g):
        return o_hbm.at[pl.ds((base + g * _CH) * _ROUT, _CH * _ROUT)]

    for b in range(2):
        pltpu.make_async_copy(in_src(jnp.int32(b)), inbufs[b], isems[b]).start()

    def pair_body(g0, carry):
        for b in range(2):
            g = 2 * g0 + b
            pltpu.make_async_copy(in_src(g), inbufs[b], isems[b]).wait()

            @pl.when(g >= 2)
            def _():
                pltpu.make_async_copy(outbufs[b], out_dst(g), osems[b]).wait()

            _compute_chunk(inbufs[b], outbufs[b], idx0)
            pltpu.make_async_copy(outbufs[b], out_dst(g), osems[b]).start()
            gn = jnp.minimum(g + 2, last)
            pltpu.make_async_copy(in_src(gn), inbufs[b], isems[b]).start()
        return carry

    lax.fori_loop(0, nch // 2 - 1, pair_body, 0)

    for b in range(2):
        g = nch - 2 + b
        pltpu.make_async_copy(in_src(g), inbufs[b], isems[b]).wait()
        pltpu.make_async_copy(outbufs[b], out_dst(g), osems[b]).wait()
        _compute_chunk(inbufs[b], outbufs[b], idx0)
        pltpu.make_async_copy(outbufs[b], out_dst(g), osems[b]).start()
    for b in range(2):
        g = nch - 2 + b
        pltpu.make_async_copy(outbufs[b], out_dst(g), osems[b]).wait()


def kernel(tensor):
    B, C, H, W = tensor.shape
    n2 = B * C * (H // 2)
    x = tensor.reshape(n2 * 2 * W)
    sc_pool = functools.partial(
        pl.kernel,
        out_type=jax.ShapeDtypeStruct((n2 * (W // 2),), jnp.float32),
        mesh=plsc.VectorSubcoreMesh(core_axis_name="c", subcore_axis_name="s"),
        compiler_params=pltpu.CompilerParams(needs_layout_passes=False),
        scratch_types=[
            pltpu.VMEM((_CH * _RIN,), jnp.float32),
            pltpu.VMEM((_CH * _RIN,), jnp.float32),
            pltpu.VMEM((_CH * _ROUT,), jnp.float32),
            pltpu.VMEM((_CH * _ROUT,), jnp.float32),
            pltpu.SemaphoreType.DMA,
            pltpu.SemaphoreType.DMA,
            pltpu.SemaphoreType.DMA,
            pltpu.SemaphoreType.DMA,
        ],
    )(_sc_pool_body)
    out = sc_pool(x)
    return out.reshape(B, C, H // 2, W // 2)


# speedup vs baseline: 4.5728x; 2.1198x over previous
import functools

import jax
import jax.numpy as jnp
from jax import lax
from jax.experimental import pallas as pl
from jax.experimental.pallas import tpu as pltpu
from jax.experimental.pallas import tpu_sc as plsc

_NC, _NS, _L = 2, 16, 16
_NW = _NC * _NS
_CHR = 32
_W = 512


def _compute_chunk(inb, outb, idx0):
    ces = [idx0 + 32 * k for k in range(16)]
    cos = [idx0 + (32 * k + 1) for k in range(16)]

    def row_body(q, carry):
        r0 = jnp.full((16,), 2 * q, jnp.int32)
        r1 = r0 + 1
        for kk in range(0, 16, 4):
            ks = range(kk, kk + 4)
            g = [[jnp.maximum(plsc.load_gather(inb, [r, c]), 0.0)
                  for r, c in ((r0, ces[k]), (r0, cos[k]),
                               (r1, ces[k]), (r1, cos[k]))]
                 for k in ks]
            dens = [(e1 + o1) + (e2 + o2) for e1, o1, e2, o2 in g]
            dens = [jnp.where(d == 0.0, 1.0, d) for d in dens]
            nums = [(e1 * e1 + o1 * o1) + (e2 * e2 + o2 * o2)
                    for e1, o1, e2, o2 in g]
            for k, num, den in zip(ks, nums, dens):
                outb[q, pl.ds(16 * k, 16)] = num / den
        return carry

    lax.fori_loop(0, _CHR // 2, row_body, 0)


def _sc_pool_body(x_hbm, o_hbm, ib0, ib1, ob0, ob1,
                  isem0, isem1, osem0, osem1):
    c = lax.axis_index("c")
    s = lax.axis_index("s")
    wid = s * _NC + c
    nplanes = x_hbm.shape[0]
    h = x_hbm.shape[1]
    planes_per_w = nplanes // _NW
    chunks_per_plane = h // _CHR
    nch = planes_per_w * chunks_per_plane
    idx0 = lax.iota(jnp.int32, 16) * 2
    inbufs = (ib0, ib1)
    outbufs = (ob0, ob1)
    isems = (isem0, isem1)
    osems = (osem0, osem1)

    def in_src(g):
        plane = wid * planes_per_w + g // chunks_per_plane
        r0 = (g % chunks_per_plane) * _CHR
        return x_hbm.at[plane, pl.ds(r0, _CHR), :]

    def out_dst(g):
        plane = wid * planes_per_w + g // chunks_per_plane
        r0 = (g % chunks_per_plane) * (_CHR // 2)
        return o_hbm.at[plane, pl.ds(r0, _CHR // 2), :]

    for b in range(2):
        pltpu.make_async_copy(in_src(jnp.int32(b)), inbufs[b], isems[b]).start()

    def pair_body(g0, carry):
        for b in range(2):
            g = 2 * g0 + b
            pltpu.make_async_copy(in_src(g), inbufs[b], isems[b]).wait()

            @pl.when(g >= 2)
            def _():
                pltpu.make_async_copy(outbufs[b], out_dst(g), osems[b]).wait()

            _compute_chunk(inbufs[b], outbufs[b], idx0)
            pltpu.make_async_copy(outbufs[b], out_dst(g), osems[b]).start()
            pltpu.make_async_copy(in_src(g + 2), inbufs[b], isems[b]).start()
        return carry

    lax.fori_loop(0, nch // 2 - 1, pair_body, 0)

    for b in range(2):
        g = nch - 2 + b
        pltpu.make_async_copy(in_src(g), inbufs[b], isems[b]).wait()
        pltpu.make_async_copy(outbufs[b], out_dst(g), osems[b]).wait()
        _compute_chunk(inbufs[b], outbufs[b], idx0)
        pltpu.make_async_copy(outbufs[b], out_dst(g), osems[b]).start()
    for b in range(2):
        g = nch - 2 + b
        pltpu.make_async_copy(outbufs[b], out_dst(g), osems[b]).wait()


def kernel(tensor):
    B, C, H, W = tensor.shape
    x = tensor.reshape(B * C, H, W)
    sc_pool = functools.partial(
        pl.kernel,
        out_type=jax.ShapeDtypeStruct((B * C, H // 2, W // 2), jnp.float32),
        mesh=plsc.VectorSubcoreMesh(core_axis_name="c", subcore_axis_name="s"),
        compiler_params=pltpu.CompilerParams(needs_layout_passes=False),
        scratch_types=[
            pltpu.VMEM((_CHR, _W), jnp.float32),
            pltpu.VMEM((_CHR, _W), jnp.float32),
            pltpu.VMEM((_CHR // 2, _W // 2), jnp.float32),
            pltpu.VMEM((_CHR // 2, _W // 2), jnp.float32),
            pltpu.SemaphoreType.DMA,
            pltpu.SemaphoreType.DMA,
            pltpu.SemaphoreType.DMA,
            pltpu.SemaphoreType.DMA,
        ],
    )(_sc_pool_body)
    out = sc_pool(x)
    return out.reshape(B, C, H // 2, W // 2)


# speedup vs baseline: 5.0425x; 1.1027x over previous
import functools

import jax
import jax.numpy as jnp
from jax import lax
from jax.experimental import pallas as pl
from jax.experimental.pallas import tpu as pltpu
from jax.experimental.pallas import tpu_sc as plsc

_NC, _NS, _L = 2, 16, 16
_NW = _NC * _NS
_CHR = 32
_W = 512


def _compute_chunk(inb, outb, idx0):
    i_e = idx0
    i_o = idx0 + 1

    def row_body(q, carry):
        for kk in range(0, 16, 4):
            ks = range(kk, kk + 4)
            wins = [(inb.at[2 * q, pl.ds(32 * k, 32)],
                     inb.at[2 * q + 1, pl.ds(32 * k, 32)]) for k in ks]
            g = [[jnp.maximum(plsc.load_gather(w, [i]), 0.0)
                  for w, i in ((we, i_e), (we, i_o), (wo, i_e), (wo, i_o))]
                 for we, wo in wins]
            dens = [(e1 + o1) + (e2 + o2) for e1, o1, e2, o2 in g]
            dens = [jnp.where(d == 0.0, 1.0, d) for d in dens]
            nums = [(e1 * e1 + o1 * o1) + (e2 * e2 + o2 * o2)
                    for e1, o1, e2, o2 in g]
            for k, num, den in zip(ks, nums, dens):
                outb[q, pl.ds(16 * k, 16)] = num / den
        return carry

    lax.fori_loop(0, _CHR // 2, row_body, 0)


def _sc_pool_body(x_hbm, o_hbm, ib0, ib1, ob0, ob1,
                  isem0, isem1, osem0, osem1):
    c = lax.axis_index("c")
    s = lax.axis_index("s")
    wid = s * _NC + c
    nplanes = x_hbm.shape[0]
    h = x_hbm.shape[1]
    planes_per_w = nplanes // _NW
    chunks_per_plane = h // _CHR
    nch = planes_per_w * chunks_per_plane
    idx0 = lax.iota(jnp.int32, 16) * 2
    inbufs = (ib0, ib1)
    outbufs = (ob0, ob1)
    isems = (isem0, isem1)
    osems = (osem0, osem1)

    def in_src(g):
        plane = wid * planes_per_w + g // chunks_per_plane
        r0 = (g % chunks_per_plane) * _CHR
        return x_hbm.at[plane, pl.ds(r0, _CHR), :]

    def out_dst(g):
        plane = wid * planes_per_w + g // chunks_per_plane
        r0 = (g % chunks_per_plane) * (_CHR // 2)
        return o_hbm.at[plane, pl.ds(r0, _CHR // 2), :]

    for b in range(2):
        pltpu.make_async_copy(in_src(jnp.int32(b)), inbufs[b], isems[b]).start()

    def pair_body(g0, carry):
        for b in range(2):
            g = 2 * g0 + b
            pltpu.make_async_copy(in_src(g), inbufs[b], isems[b]).wait()

            @pl.when(g >= 2)
            def _():
                pltpu.make_async_copy(outbufs[b], out_dst(g), osems[b]).wait()

            _compute_chunk(inbufs[b], outbufs[b], idx0)
            pltpu.make_async_copy(outbufs[b], out_dst(g), osems[b]).start()
            pltpu.make_async_copy(in_src(g + 2), inbufs[b], isems[b]).start()
        return carry

    lax.fori_loop(0, nch // 2 - 1, pair_body, 0)

    for b in range(2):
        g = nch - 2 + b
        pltpu.make_async_copy(in_src(g), inbufs[b], isems[b]).wait()
        pltpu.make_async_copy(outbufs[b], out_dst(g), osems[b]).wait()
        _compute_chunk(inbufs[b], outbufs[b], idx0)
        pltpu.make_async_copy(outbufs[b], out_dst(g), osems[b]).start()
    for b in range(2):
        g = nch - 2 + b
        pltpu.make_async_copy(outbufs[b], out_dst(g), osems[b]).wait()


def kernel(tensor):
    B, C, H, W = tensor.shape
    x = tensor.reshape(B * C, H, W)
    sc_pool = functools.partial(
        pl.kernel,
        out_type=jax.ShapeDtypeStruct((B * C, H // 2, W // 2), jnp.float32),
        mesh=plsc.VectorSubcoreMesh(core_axis_name="c", subcore_axis_name="s"),
        compiler_params=pltpu.CompilerParams(needs_layout_passes=False),
        scratch_types=[
            pltpu.VMEM((_CHR, _W), jnp.float32),
            pltpu.VMEM((_CHR, _W), jnp.float32),
            pltpu.VMEM((_CHR // 2, _W // 2), jnp.float32),
            pltpu.VMEM((_CHR // 2, _W // 2), jnp.float32),
            pltpu.SemaphoreType.DMA,
            pltpu.SemaphoreType.DMA,
            pltpu.SemaphoreType.DMA,
            pltpu.SemaphoreType.DMA,
        ],
    )(_sc_pool_body)
    out = sc_pool(x)
    return out.reshape(B, C, H // 2, W // 2)


# speedup vs baseline: 5.0713x; 1.0057x over previous
import functools

import jax
import jax.numpy as jnp
from jax import lax
from jax.experimental import pallas as pl
from jax.experimental.pallas import tpu as pltpu
from jax.experimental.pallas import tpu_sc as plsc

_NC, _NS, _L = 2, 16, 16
_NW = _NC * _NS
_CHR = 64
_W = 512


def _compute_chunk(inb, outb, idx0):
    i_e = idx0
    i_o = idx0 + 1

    def row_body(q, carry):
        for kk in range(0, 16, 4):
            ks = range(kk, kk + 4)
            wins = [(inb.at[2 * q, pl.ds(32 * k, 32)],
                     inb.at[2 * q + 1, pl.ds(32 * k, 32)]) for k in ks]
            g = [[jnp.maximum(plsc.load_gather(w, [i]), 0.0)
                  for w, i in ((we, i_e), (we, i_o), (wo, i_e), (wo, i_o))]
                 for we, wo in wins]
            dens = [(e1 + o1) + (e2 + o2) for e1, o1, e2, o2 in g]
            dens = [jnp.where(d == 0.0, 1.0, d) for d in dens]
            nums = [(e1 * e1 + o1 * o1) + (e2 * e2 + o2 * o2)
                    for e1, o1, e2, o2 in g]
            for k, num, den in zip(ks, nums, dens):
                outb[q, pl.ds(16 * k, 16)] = num / den
        return carry

    lax.fori_loop(0, _CHR // 2, row_body, 0)


def _sc_pool_body(x_hbm, o_hbm, ib0, ib1, ob0, ob1,
                  isem0, isem1, osem0, osem1):
    c = lax.axis_index("c")
    s = lax.axis_index("s")
    wid = s * _NC + c
    nplanes = x_hbm.shape[0]
    h = x_hbm.shape[1]
    planes_per_w = nplanes // _NW
    chunks_per_plane = h // _CHR
    nch = planes_per_w * chunks_per_plane
    idx0 = lax.iota(jnp.int32, 16) * 2
    inbufs = (ib0, ib1)
    outbufs = (ob0, ob1)
    isems = (isem0, isem1)
    osems = (osem0, osem1)

    def in_src(g):
        plane = wid * planes_per_w + g // chunks_per_plane
        r0 = (g % chunks_per_plane) * _CHR
        return x_hbm.at[plane, pl.ds(r0, _CHR), :]

    def out_dst(g):
        plane = wid * planes_per_w + g // chunks_per_plane
        r0 = (g % chunks_per_plane) * (_CHR // 2)
        return o_hbm.at[plane, pl.ds(r0, _CHR // 2), :]

    for b in range(2):
        pltpu.make_async_copy(in_src(jnp.int32(b)), inbufs[b], isems[b]).start()

    def pair_body(g0, carry):
        for b in range(2):
            g = 2 * g0 + b
            pltpu.make_async_copy(in_src(g), inbufs[b], isems[b]).wait()

            @pl.when(g >= 2)
            def _():
                pltpu.make_async_copy(outbufs[b], out_dst(g), osems[b]).wait()

            _compute_chunk(inbufs[b], outbufs[b], idx0)
            pltpu.make_async_copy(outbufs[b], out_dst(g), osems[b]).start()
            pltpu.make_async_copy(in_src(g + 2), inbufs[b], isems[b]).start()
        return carry

    lax.fori_loop(0, nch // 2 - 1, pair_body, 0)

    for b in range(2):
        g = nch - 2 + b
        pltpu.make_async_copy(in_src(g), inbufs[b], isems[b]).wait()
        pltpu.make_async_copy(outbufs[b], out_dst(g), osems[b]).wait()
        _compute_chunk(inbufs[b], outbufs[b], idx0)
        pltpu.make_async_copy(outbufs[b], out_dst(g), osems[b]).start()
    for b in range(2):
        g = nch - 2 + b
        pltpu.make_async_copy(outbufs[b], out_dst(g), osems[b]).wait()


def kernel(tensor):
    B, C, H, W = tensor.shape
    x = tensor.reshape(B * C, H, W)
    sc_pool = functools.partial(
        pl.kernel,
        out_type=jax.ShapeDtypeStruct((B * C, H // 2, W // 2), jnp.float32),
        mesh=plsc.VectorSubcoreMesh(core_axis_name="c", subcore_axis_name="s"),
        compiler_params=pltpu.CompilerParams(needs_layout_passes=False),
        scratch_types=[
            pltpu.VMEM((_CHR, _W), jnp.float32),
            pltpu.VMEM((_CHR, _W), jnp.float32),
            pltpu.VMEM((_CHR // 2, _W // 2), jnp.float32),
            pltpu.VMEM((_CHR // 2, _W // 2), jnp.float32),
            pltpu.SemaphoreType.DMA,
            pltpu.SemaphoreType.DMA,
            pltpu.SemaphoreType.DMA,
            pltpu.SemaphoreType.DMA,
        ],
    )(_sc_pool_body)
    out = sc_pool(x)
    return out.reshape(B, C, H // 2, W // 2)


# speedup vs baseline: 6.5059x; 1.2829x over previous
import functools

import jax
import jax.numpy as jnp
from jax import lax
from jax.experimental import pallas as pl
from jax.experimental.pallas import tpu as pltpu
from jax.experimental.pallas import tpu_sc as plsc

_NC, _NS, _L = 2, 16, 16
_NW = _NC * _NS
_CHR = 64
_W = 512


def _compute_chunk(inb, outb, idx0):
    i_e = idx0
    i_o = idx0 + 1

    def row_body(q, carry):
        for kk in range(0, 16, 8):
            ks = range(kk, kk + 8)
            wins = [(inb.at[2 * q, pl.ds(32 * k, 32)],
                     inb.at[2 * q + 1, pl.ds(32 * k, 32)]) for k in ks]
            g = [[jnp.maximum(plsc.load_gather(w, [i]), 0.0)
                  for w, i in ((we, i_e), (we, i_o), (wo, i_e), (wo, i_o))]
                 for we, wo in wins]
            dens = [jnp.maximum((e1 + o1) + (e2 + o2), 1.2e-38)
                    for e1, o1, e2, o2 in g]
            nums = [(e1 * e1 + o1 * o1) + (e2 * e2 + o2 * o2)
                    for e1, o1, e2, o2 in g]
            for k, num, den in zip(ks, nums, dens):
                outb[q, pl.ds(16 * k, 16)] = num / den
        return carry

    lax.fori_loop(0, _CHR // 2, row_body, 0)


def _sc_pool_body(x_hbm, o_hbm, ib0, ib1, ob0, ob1,
                  isem0, isem1, osem0, osem1):
    c = lax.axis_index("c")
    s = lax.axis_index("s")
    wid = s * _NC + c
    nplanes = x_hbm.shape[0]
    h = x_hbm.shape[1]
    planes_per_w = nplanes // _NW
    chunks_per_plane = h // _CHR
    nch = planes_per_w * chunks_per_plane
    idx0 = lax.iota(jnp.int32, 16) * 2
    inbufs = (ib0, ib1)
    outbufs = (ob0, ob1)
    isems = (isem0, isem1)
    osems = (osem0, osem1)

    def in_src(g):
        plane = wid * planes_per_w + g // chunks_per_plane
        r0 = (g % chunks_per_plane) * _CHR
        return x_hbm.at[plane, pl.ds(r0, _CHR), :]

    def out_dst(g):
        plane = wid * planes_per_w + g // chunks_per_plane
        r0 = (g % chunks_per_plane) * (_CHR // 2)
        return o_hbm.at[plane, pl.ds(r0, _CHR // 2), :]

    for b in range(2):
        pltpu.make_async_copy(in_src(jnp.int32(b)), inbufs[b], isems[b]).start()

    def pair_body(g0, carry):
        for b in range(2):
            g = 2 * g0 + b
            pltpu.make_async_copy(in_src(g), inbufs[b], isems[b]).wait()

            @pl.when(g >= 2)
            def _():
                pltpu.make_async_copy(outbufs[b], out_dst(g), osems[b]).wait()

            _compute_chunk(inbufs[b], outbufs[b], idx0)
            pltpu.make_async_copy(outbufs[b], out_dst(g), osems[b]).start()
            pltpu.make_async_copy(in_src(g + 2), inbufs[b], isems[b]).start()
        return carry

    lax.fori_loop(0, nch // 2 - 1, pair_body, 0)

    for b in range(2):
        g = nch - 2 + b
        pltpu.make_async_copy(in_src(g), inbufs[b], isems[b]).wait()
        pltpu.make_async_copy(outbufs[b], out_dst(g), osems[b]).wait()
        _compute_chunk(inbufs[b], outbufs[b], idx0)
        pltpu.make_async_copy(outbufs[b], out_dst(g), osems[b]).start()
    for b in range(2):
        g = nch - 2 + b
        pltpu.make_async_copy(outbufs[b], out_dst(g), osems[b]).wait()


def kernel(tensor):
    B, C, H, W = tensor.shape
    x = tensor.reshape(B * C, H, W)
    sc_pool = functools.partial(
        pl.kernel,
        out_type=jax.ShapeDtypeStruct((B * C, H // 2, W // 2), jnp.float32),
        mesh=plsc.VectorSubcoreMesh(core_axis_name="c", subcore_axis_name="s"),
        compiler_params=pltpu.CompilerParams(needs_layout_passes=False),
        scratch_types=[
            pltpu.VMEM((_CHR, _W), jnp.float32),
            pltpu.VMEM((_CHR, _W), jnp.float32),
            pltpu.VMEM((_CHR // 2, _W // 2), jnp.float32),
            pltpu.VMEM((_CHR // 2, _W // 2), jnp.float32),
            pltpu.SemaphoreType.DMA,
            pltpu.SemaphoreType.DMA,
            pltpu.SemaphoreType.DMA,
            pltpu.SemaphoreType.DMA,
        ],
    )(_sc_pool_body)
    out = sc_pool(x)
    return out.reshape(B, C, H // 2, W // 2)
